# Initial kernel scaffold; baseline (speedup 1.0000x reference)
import jax
import jax.numpy as jnp
from jax.experimental import pallas as pl


def kernel(node_features, edge_index, edge_timestamps, target_nodes, W_t1, b_t1, W_t2, b_t2, W_f, b_f, W_in, b_in, W_out, b_out, W_proj, b_proj):
    raise NotImplementedError("write your pallas kernel here")


# speedup vs baseline: 3.4899x; 3.4899x over previous
import jax
import jax.numpy as jnp
from jax.experimental import pallas as pl


_K = 2048
_NH = 4
_E2 = 1 << 18


def _dense_kernel(nbr_f_ref, ts_ref, valid_ref, tgt_ref,
                  wf_ref, bf_ref, w1_ref, b1_ref, wt2_ref, bt2_ref,
                  wq_ref, bq_ref, wk_ref, bk_ref, wv_ref, bv_ref,
                  wo_ref, bo_ref, wp_ref, bp_ref,
                  out_ref):
    K = nbr_f_ref.shape[1]
    H = wf_ref.shape[1]
    dh = H // _NH

    nbr_f = nbr_f_ref[...].reshape(K, nbr_f_ref.shape[2])
    ts = ts_ref[...].reshape(K, 1)
    valid = valid_ref[...].reshape(K, 1)

    bf = bf_ref[...]
    t_nbr = jnp.dot(nbr_f, wf_ref[...],
                    preferred_element_type=jnp.float32) + bf
    r = jnp.maximum(ts * w1_ref[...] + b1_ref[...], 0.0)
    kv = t_nbr + jnp.dot(r, wt2_ref[...],
                         preferred_element_type=jnp.float32) + bt2_ref[...]

    tgt_tr = jnp.dot(tgt_ref[...].reshape(1, nbr_f_ref.shape[2]), wf_ref[...],
                     preferred_element_type=jnp.float32) + bf
    q = jnp.dot(tgt_tr, wq_ref[...],
                preferred_element_type=jnp.float32) + bq_ref[...]
    kmat = jnp.dot(kv, wk_ref[...],
                   preferred_element_type=jnp.float32) + bk_ref[...]
    vmat = jnp.dot(kv, wv_ref[...],
                   preferred_element_type=jnp.float32) + bv_ref[...]

    prod = kmat * q
    scale = 1.0 / (float(dh) ** 0.5)
    logit_parts = [
        jnp.sum(prod[:, h * dh:(h + 1) * dh], axis=1, keepdims=True) * scale
        for h in range(_NH)
    ]
    logits = jnp.concatenate(logit_parts, axis=1)
    logits = logits + (valid - 1.0) * 1e9

    mx = jnp.max(logits, axis=0, keepdims=True)
    ex = jnp.exp(logits - mx)
    sm = jnp.sum(ex, axis=0, keepdims=True)
    attn = ex / sm

    ctx_parts = [
        jnp.sum(attn[:, h:h + 1] * vmat[:, h * dh:(h + 1) * dh],
                axis=0, keepdims=True)
        for h in range(_NH)
    ]
    ctx = jnp.concatenate(ctx_parts, axis=1)
    attended = jnp.dot(ctx, wo_ref[...],
                       preferred_element_type=jnp.float32) + bo_ref[...]
    has_nbr = jnp.max(valid) > 0.0
    agg = jnp.where(has_nbr, attended, tgt_tr)
    res = jnp.dot(agg, wp_ref[...],
                  preferred_element_type=jnp.float32) + bp_ref[...]
    out_ref[...] = res.reshape(1, 1, H)


def kernel(node_features, edge_index, edge_timestamps, target_nodes,
           W_t1, b_t1, W_t2, b_t2, W_f, b_f, W_in, b_in,
           W_out, b_out, W_proj, b_proj):
    N, F = node_features.shape
    E = edge_index.shape[1]
    B = target_nodes.shape[0]
    H = W_f.shape[0]
    K = _K

    src = edge_index[0]
    dst = edge_index[1]
    bidx = jnp.arange(B, dtype=jnp.int32)

    slot_of = jnp.full((N,), B, jnp.int32).at[target_nodes].min(bidx)
    ss = slot_of[src]
    sd = slot_of[dst]
    e = jnp.arange(E, dtype=jnp.int32)
    big = jnp.int32(B * _E2)
    ks = jnp.where(ss < B, ss * _E2 + e, big)
    kd = jnp.where((sd < B) & (src != dst), sd * _E2 + e, big)
    srt = jnp.sort(jnp.concatenate([ks, kd]))

    starts = jnp.searchsorted(srt, bidx * _E2).astype(jnp.int32)
    ends = jnp.searchsorted(srt, (bidx + 1) * _E2).astype(jnp.int32)
    cnt = jnp.minimum(ends - starts, K)

    j = jnp.arange(K, dtype=jnp.int32)
    pos = jnp.minimum(starts[:, None] + j[None, :], 2 * E - 1)
    key = srt[pos]
    validu = j[None, :] < cnt[:, None]
    eid_u = jnp.where(validu, key - bidx[:, None] * _E2, 0)

    s_b = slot_of[target_nodes]
    eid = eid_u[s_b]
    valid = validu[s_b].astype(jnp.float32)
    t_b = target_nodes

    src_e = src[eid]
    dst_e = dst[eid]
    nbr = jnp.where(src_e == t_b[:, None], dst_e, src_e)
    nbr = jnp.where(valid > 0, nbr, 0)

    ts_g = edge_timestamps[eid][:, :, None]
    nbr_feat = node_features[nbr]
    tgt_feat = node_features[t_b][:, None, :]
    valid3 = valid[:, :, None]

    Wq, Wk, Wv = jnp.split(W_in, 3, axis=0)
    bq, bk, bv = jnp.split(b_in, 3)
    wf = W_f.T
    w1 = W_t1.T
    args = (
        nbr_feat, ts_g, valid3, tgt_feat,
        wf, b_f[None, :], w1, b_t1[None, :], W_t2.T, b_t2[None, :],
        Wq.T, bq[None, :], Wk.T, bk[None, :], Wv.T, bv[None, :],
        W_out.T, b_out[None, :], W_proj.T, b_proj[None, :],
    )

    def w_spec(shape):
        nd = len(shape)
        return pl.BlockSpec(shape, lambda b, _n=nd: (0,) * _n)

    in_specs = [
        pl.BlockSpec((1, K, F), lambda b: (b, 0, 0)),
        pl.BlockSpec((1, K, 1), lambda b: (b, 0, 0)),
        pl.BlockSpec((1, K, 1), lambda b: (b, 0, 0)),
        pl.BlockSpec((1, 1, F), lambda b: (b, 0, 0)),
        w_spec((F, H)), w_spec((1, H)),
        w_spec((1, H)), w_spec((1, H)),
        w_spec((H, H)), w_spec((1, H)),
        w_spec((H, H)), w_spec((1, H)),
        w_spec((H, H)), w_spec((1, H)),
        w_spec((H, H)), w_spec((1, H)),
        w_spec((H, H)), w_spec((1, H)),
        w_spec((H, H)), w_spec((1, H)),
    ]

    out = pl.pallas_call(
        _dense_kernel,
        grid=(B,),
        in_specs=in_specs,
        out_specs=pl.BlockSpec((1, 1, H), lambda b: (b, 0, 0)),
        out_shape=jax.ShapeDtypeStruct((B, 1, H), jnp.float32),
    )(*args)
    return out.reshape(B, H)
